# baseline (speedup 1.0000x reference)
import functools
import math

import jax
import jax.numpy as jnp
from jax import lax
from jax.experimental import pallas as pl
from jax.experimental.pallas import tpu as pltpu
from jax.experimental.pallas import tpu_sc as plsc

_BATCH = 4096
_N = 100000
_D = 64
_TEMP = 0.05
_BK = 4096
_SUB = 2048
_NK = (_N + _BK - 1) // _BK
_NPAD = _NK * _BK
_PAD = _NPAD - _N
_SCALE = math.log2(math.e) / _TEMP

_NW = 32
_BPW = _BATCH // _NW


def _sc_gather(indexes, labels, packed):
    mesh = plsc.VectorSubcoreMesh(core_axis_name="c", subcore_axis_name="s")

    @functools.partial(
        pl.kernel,
        mesh=mesh,
        out_type=(
            jax.ShapeDtypeStruct((_BATCH,), jnp.int32),
            jax.ShapeDtypeStruct((_BATCH, 2 * _D), jnp.float32),
        ),
        scratch_types=[
            pltpu.VMEM((_BPW,), jnp.int32),
            pltpu.VMEM((_BPW,), jnp.int32),
            pltpu.VMEM((_BPW,), jnp.int32),
            pltpu.VMEM((_BPW, 2 * _D), jnp.float32),
            pltpu.SemaphoreType.DMA,
            pltpu.SemaphoreType.DMA,
        ],
    )
    def k(idx_hbm, lab_hbm, pk_hbm, tgt_out, rows_out,
          idx_v, tgt_v, q_v, rows_v, sem1, sem2):
        wid = lax.axis_index("s") * 2 + lax.axis_index("c")
        base = wid * _BPW
        pltpu.sync_copy(idx_hbm.at[pl.ds(base, _BPW)], idx_v)
        pltpu.async_copy(lab_hbm.at[idx_v], tgt_v, sem1).wait()
        for i in range(_BPW // 16):
            sl = pl.ds(i * 16, 16)
            t = tgt_v[sl]
            q_v[sl] = jnp.bitwise_or(
                lax.shift_left(lax.shift_right_logical(t, 11), 10),
                jnp.bitwise_and(t, 1023),
            )
        pltpu.async_copy(pk_hbm.at[q_v], rows_v, sem2).wait()
        pltpu.sync_copy(tgt_v, tgt_out.at[pl.ds(base, _BPW)])
        pltpu.sync_copy(rows_v, rows_out.at[pl.ds(base, _BPW)])

    return k(indexes, labels, packed)


def _sums_kernel(x_ref, ct_ref, sums_ref, pk_ref):
    k = pl.program_id(0)

    @pl.when(k == 0)
    def _init():
        sums_ref[...] = jnp.zeros_like(sums_ref)

    x = x_ref[...]
    acc = jnp.zeros((_BATCH, 1), jnp.float32)
    for b in range(_BK // _SUB):
        limit = _N - k * _BK - b * _SUB
        cid = jax.lax.broadcasted_iota(jnp.int32, (_D, _SUB), 1)
        craw = ct_ref[:, b * _SUB : (b + 1) * _SUB]
        c = jnp.where(cid < limit, craw, 0.0).astype(jnp.bfloat16)
        y = jax.lax.dot_general(
            x, c, (((1,), (0,)), ((), ())),
            preferred_element_type=jnp.float32,
        )
        e1 = jnp.exp2(y[:, : _SUB // 2])
        e2 = jnp.exp2(y[:, _SUB // 2 :].astype(jnp.bfloat16)).astype(jnp.float32)
        acc = acc + jnp.sum(e1, axis=1, keepdims=True)
        acc = acc + jnp.sum(e2, axis=1, keepdims=True)
        crows = craw.T
        pk_ref[b * (_SUB // 2) : (b + 1) * (_SUB // 2), :] = jnp.concatenate(
            [crows[: _SUB // 2], crows[_SUB // 2 :]], axis=1
        )
    sums_ref[...] += acc


def _finish_kernel(x_ref, rows_ref, tgt_ref, sums_ref, loss_ref):
    x = x_ref[...]
    p0 = jnp.sum(x * rows_ref[:, :_D], axis=1, keepdims=True)
    p1 = jnp.sum(x * rows_ref[:, _D:], axis=1, keepdims=True)
    hi = jnp.bitwise_and(jax.lax.shift_right_logical(tgt_ref[...], 10), 1) == 1
    p = jnp.where(hi, p1, p0)
    s = sums_ref[...] - float(_PAD)
    lp = jnp.log(jnp.exp2(p) / (s + 1e-6) + 1e-6)
    loss_ref[...] = jnp.sum(lp, axis=0, keepdims=True) * (-1.0 / _BATCH)


@jax.jit
def kernel(inputs, indexes, labels, instance_features, cluster_features):
    del instance_features
    xs = inputs * jnp.float32(_SCALE)
    xb = xs.astype(jnp.bfloat16)
    ct = cluster_features.T
    sums, packed = pl.pallas_call(
        _sums_kernel,
        grid=(_NK,),
        in_specs=[
            pl.BlockSpec((_BATCH, _D), lambda k: (0, 0)),
            pl.BlockSpec((_D, _BK), lambda k: (0, k)),
        ],
        out_specs=[
            pl.BlockSpec((_BATCH, 1), lambda k: (0, 0)),
            pl.BlockSpec((_BK // 2, 2 * _D), lambda k: (k, 0)),
        ],
        out_shape=[
            jax.ShapeDtypeStruct((_BATCH, 1), jnp.float32),
            jax.ShapeDtypeStruct((_NK * _BK // 2, 2 * _D), jnp.float32),
        ],
        compiler_params=pltpu.CompilerParams(
            dimension_semantics=("arbitrary",),
        ),
    )(xb, ct)
    tgt, rows = _sc_gather(
        indexes.astype(jnp.int32), labels.astype(jnp.int32), packed
    )
    loss = pl.pallas_call(
        _finish_kernel,
        in_specs=[
            pl.BlockSpec((_BATCH, _D), lambda: (0, 0)),
            pl.BlockSpec((_BATCH, 2 * _D), lambda: (0, 0)),
            pl.BlockSpec((_BATCH, 1), lambda: (0, 0)),
            pl.BlockSpec((_BATCH, 1), lambda: (0, 0)),
        ],
        out_specs=pl.BlockSpec((1, 1), lambda: (0, 0)),
        out_shape=jax.ShapeDtypeStruct((1, 1), jnp.float32),
    )(xs, rows, tgt.reshape(_BATCH, 1), sums)
    return loss[0, 0]


# speedup vs baseline: 1.0007x; 1.0007x over previous
import functools
import math

import jax
import jax.numpy as jnp
from jax import lax
from jax.experimental import pallas as pl
from jax.experimental.pallas import tpu as pltpu
from jax.experimental.pallas import tpu_sc as plsc

_BATCH = 4096
_N = 100000
_D = 64
_TEMP = 0.05
_BK = 2048
_SUB = 2048
_NK = (_N + _BK - 1) // _BK
_NPAD = _NK * _BK
_PAD = _NPAD - _N
_SCALE = math.log2(math.e) / _TEMP

_NW = 32
_BPW = _BATCH // _NW


def _sc_gather(indexes, labels, packed):
    mesh = plsc.VectorSubcoreMesh(core_axis_name="c", subcore_axis_name="s")

    @functools.partial(
        pl.kernel,
        mesh=mesh,
        out_type=(
            jax.ShapeDtypeStruct((_BATCH,), jnp.int32),
            jax.ShapeDtypeStruct((_BATCH, 2 * _D), jnp.float32),
        ),
        scratch_types=[
            pltpu.VMEM((_BPW,), jnp.int32),
            pltpu.VMEM((_BPW,), jnp.int32),
            pltpu.VMEM((_BPW,), jnp.int32),
            pltpu.VMEM((_BPW, 2 * _D), jnp.float32),
            pltpu.SemaphoreType.DMA,
            pltpu.SemaphoreType.DMA,
        ],
    )
    def k(idx_hbm, lab_hbm, pk_hbm, tgt_out, rows_out,
          idx_v, tgt_v, q_v, rows_v, sem1, sem2):
        wid = lax.axis_index("s") * 2 + lax.axis_index("c")
        base = wid * _BPW
        pltpu.sync_copy(idx_hbm.at[pl.ds(base, _BPW)], idx_v)
        pltpu.async_copy(lab_hbm.at[idx_v], tgt_v, sem1).wait()
        for i in range(_BPW // 16):
            sl = pl.ds(i * 16, 16)
            t = tgt_v[sl]
            q_v[sl] = jnp.bitwise_or(
                lax.shift_left(lax.shift_right_logical(t, 11), 10),
                jnp.bitwise_and(t, 1023),
            )
        pltpu.async_copy(pk_hbm.at[q_v], rows_v, sem2).wait()
        pltpu.sync_copy(tgt_v, tgt_out.at[pl.ds(base, _BPW)])
        pltpu.sync_copy(rows_v, rows_out.at[pl.ds(base, _BPW)])

    return k(indexes, labels, packed)


def _sums_kernel(x_ref, ct_ref, sums_ref, pk_ref):
    k = pl.program_id(0)

    @pl.when(k == 0)
    def _init():
        sums_ref[...] = jnp.zeros_like(sums_ref)

    x = x_ref[...]
    acc = jnp.zeros((_BATCH, 1), jnp.float32)
    for b in range(_BK // _SUB):
        limit = _N - k * _BK - b * _SUB
        cid = jax.lax.broadcasted_iota(jnp.int32, (_D, _SUB), 1)
        craw = ct_ref[:, b * _SUB : (b + 1) * _SUB]
        c = jnp.where(cid < limit, craw, 0.0).astype(jnp.bfloat16)
        y = jax.lax.dot_general(
            x, c, (((1,), (0,)), ((), ())),
            preferred_element_type=jnp.float32,
        )
        acc = acc + jnp.sum(jnp.exp2(y), axis=1, keepdims=True)
        crows = craw.T
        pk_ref[b * (_SUB // 2) : (b + 1) * (_SUB // 2), :] = jnp.concatenate(
            [crows[: _SUB // 2], crows[_SUB // 2 :]], axis=1
        )
    sums_ref[...] += acc


def _finish_kernel(x_ref, rows_ref, tgt_ref, sums_ref, loss_ref):
    x = x_ref[...]
    p0 = jnp.sum(x * rows_ref[:, :_D], axis=1, keepdims=True)
    p1 = jnp.sum(x * rows_ref[:, _D:], axis=1, keepdims=True)
    hi = jnp.bitwise_and(jax.lax.shift_right_logical(tgt_ref[...], 10), 1) == 1
    p = jnp.where(hi, p1, p0)
    s = sums_ref[...] - float(_PAD)
    lp = jnp.log(jnp.exp2(p) / (s + 1e-6) + 1e-6)
    loss_ref[...] = jnp.sum(lp, axis=0, keepdims=True) * (-1.0 / _BATCH)


@jax.jit
def kernel(inputs, indexes, labels, instance_features, cluster_features):
    del instance_features
    xs = inputs * jnp.float32(_SCALE)
    xb = xs.astype(jnp.bfloat16)
    ct = cluster_features.T
    sums, packed = pl.pallas_call(
        _sums_kernel,
        grid=(_NK,),
        in_specs=[
            pl.BlockSpec((_BATCH, _D), lambda k: (0, 0)),
            pl.BlockSpec((_D, _BK), lambda k: (0, k)),
        ],
        out_specs=[
            pl.BlockSpec((_BATCH, 1), lambda k: (0, 0)),
            pl.BlockSpec((_BK // 2, 2 * _D), lambda k: (k, 0)),
        ],
        out_shape=[
            jax.ShapeDtypeStruct((_BATCH, 1), jnp.float32),
            jax.ShapeDtypeStruct((_NK * _BK // 2, 2 * _D), jnp.float32),
        ],
        compiler_params=pltpu.CompilerParams(
            dimension_semantics=("arbitrary",),
        ),
    )(xb, ct)
    tgt, rows = _sc_gather(
        indexes.astype(jnp.int32), labels.astype(jnp.int32), packed
    )
    loss = pl.pallas_call(
        _finish_kernel,
        in_specs=[
            pl.BlockSpec((_BATCH, _D), lambda: (0, 0)),
            pl.BlockSpec((_BATCH, 2 * _D), lambda: (0, 0)),
            pl.BlockSpec((_BATCH, 1), lambda: (0, 0)),
            pl.BlockSpec((_BATCH, 1), lambda: (0, 0)),
        ],
        out_specs=pl.BlockSpec((1, 1), lambda: (0, 0)),
        out_shape=jax.ShapeDtypeStruct((1, 1), jnp.float32),
    )(xs, rows, tgt.reshape(_BATCH, 1), sums)
    return loss[0, 0]


# speedup vs baseline: 1.0015x; 1.0008x over previous
import functools
import math

import jax
import jax.numpy as jnp
from jax import lax
from jax.experimental import pallas as pl
from jax.experimental.pallas import tpu as pltpu
from jax.experimental.pallas import tpu_sc as plsc

_BATCH = 4096
_N = 100000
_D = 64
_TEMP = 0.05
_BK = 2048
_SUB = 2048
_NK = (_N + _BK - 1) // _BK
_NPAD = _NK * _BK
_PAD = _NPAD - _N
_SCALE = math.log2(math.e) / _TEMP

_NW = 32
_BPW = _BATCH // _NW


def _sc_gather(indexes, labels, packed):
    mesh = plsc.VectorSubcoreMesh(core_axis_name="c", subcore_axis_name="s")

    @functools.partial(
        pl.kernel,
        mesh=mesh,
        out_type=(
            jax.ShapeDtypeStruct((_BATCH,), jnp.int32),
            jax.ShapeDtypeStruct((_BATCH, 2 * _D), jnp.float32),
        ),
        scratch_types=[
            pltpu.VMEM((_BPW,), jnp.int32),
            pltpu.VMEM((_BPW,), jnp.int32),
            pltpu.VMEM((_BPW,), jnp.int32),
            pltpu.VMEM((_BPW, 2 * _D), jnp.float32),
            pltpu.SemaphoreType.DMA,
            pltpu.SemaphoreType.DMA,
        ],
    )
    def k(idx_hbm, lab_hbm, pk_hbm, tgt_out, rows_out,
          idx_v, tgt_v, q_v, rows_v, sem1, sem2):
        wid = lax.axis_index("s") * 2 + lax.axis_index("c")
        base = wid * _BPW
        pltpu.sync_copy(idx_hbm.at[pl.ds(base, _BPW)], idx_v)
        pltpu.async_copy(lab_hbm.at[idx_v], tgt_v, sem1).wait()
        for i in range(_BPW // 16):
            sl = pl.ds(i * 16, 16)
            t = tgt_v[sl]
            q_v[sl] = jnp.bitwise_or(
                lax.shift_left(lax.shift_right_logical(t, 11), 10),
                jnp.bitwise_and(t, 1023),
            )
        pltpu.async_copy(pk_hbm.at[q_v], rows_v, sem2).wait()
        pltpu.sync_copy(tgt_v, tgt_out.at[pl.ds(base, _BPW)])
        pltpu.sync_copy(rows_v, rows_out.at[pl.ds(base, _BPW)])

    return k(indexes, labels, packed)


def _sums_kernel(x_ref, ct_ref, sums_ref, pk_ref):
    k = pl.program_id(0)

    @pl.when(k == 0)
    def _init():
        sums_ref[...] = jnp.zeros_like(sums_ref)

    x = (x_ref[...] * jnp.float32(_SCALE)).astype(jnp.bfloat16)
    acc = jnp.zeros((_BATCH, 1), jnp.float32)
    for b in range(_BK // _SUB):
        limit = _N - k * _BK - b * _SUB
        cid = jax.lax.broadcasted_iota(jnp.int32, (_D, _SUB), 1)
        craw = ct_ref[:, b * _SUB : (b + 1) * _SUB]
        c = jnp.where(cid < limit, craw, 0.0).astype(jnp.bfloat16)
        y = jax.lax.dot_general(
            x, c, (((1,), (0,)), ((), ())),
            preferred_element_type=jnp.float32,
        )
        acc = acc + jnp.sum(jnp.exp2(y), axis=1, keepdims=True)
        crows = craw.T
        pk_ref[b * (_SUB // 2) : (b + 1) * (_SUB // 2), :] = jnp.concatenate(
            [crows[: _SUB // 2], crows[_SUB // 2 :]], axis=1
        )
    sums_ref[...] += acc


def _finish_kernel(x_ref, rows_ref, tgt_ref, sums_ref, loss_ref):
    x = x_ref[...] * jnp.float32(_SCALE)
    p0 = jnp.sum(x * rows_ref[:, :_D], axis=1, keepdims=True)
    p1 = jnp.sum(x * rows_ref[:, _D:], axis=1, keepdims=True)
    hi = jnp.bitwise_and(jax.lax.shift_right_logical(tgt_ref[...], 10), 1) == 1
    p = jnp.where(hi, p1, p0)
    s = sums_ref[...] - float(_PAD)
    lp = jnp.log(jnp.exp2(p) / (s + 1e-6) + 1e-6)
    loss_ref[...] = jnp.sum(lp, axis=0, keepdims=True) * (-1.0 / _BATCH)


@jax.jit
def kernel(inputs, indexes, labels, instance_features, cluster_features):
    del instance_features
    ct = cluster_features.T
    sums, packed = pl.pallas_call(
        _sums_kernel,
        grid=(_NK,),
        in_specs=[
            pl.BlockSpec((_BATCH, _D), lambda k: (0, 0)),
            pl.BlockSpec((_D, _BK), lambda k: (0, k)),
        ],
        out_specs=[
            pl.BlockSpec((_BATCH, 1), lambda k: (0, 0)),
            pl.BlockSpec((_BK // 2, 2 * _D), lambda k: (k, 0)),
        ],
        out_shape=[
            jax.ShapeDtypeStruct((_BATCH, 1), jnp.float32),
            jax.ShapeDtypeStruct((_NK * _BK // 2, 2 * _D), jnp.float32),
        ],
        compiler_params=pltpu.CompilerParams(
            dimension_semantics=("arbitrary",),
        ),
    )(inputs, ct)
    tgt, rows = _sc_gather(
        indexes.astype(jnp.int32), labels.astype(jnp.int32), packed
    )
    loss = pl.pallas_call(
        _finish_kernel,
        in_specs=[
            pl.BlockSpec((_BATCH, _D), lambda: (0, 0)),
            pl.BlockSpec((_BATCH, 2 * _D), lambda: (0, 0)),
            pl.BlockSpec((_BATCH, 1), lambda: (0, 0)),
            pl.BlockSpec((_BATCH, 1), lambda: (0, 0)),
        ],
        out_specs=pl.BlockSpec((1, 1), lambda: (0, 0)),
        out_shape=jax.ShapeDtypeStruct((1, 1), jnp.float32),
    )(inputs, rows, tgt.reshape(_BATCH, 1), sums)
    return loss[0, 0]
